# Initial kernel scaffold; baseline (speedup 1.0000x reference)
import jax
import jax.numpy as jnp
from jax.experimental import pallas as pl


def kernel(rpn_class, rpn_bbox, anchors):
    raise NotImplementedError("write your pallas kernel here")


# speedup vs baseline: 20.6240x; 20.6240x over previous
import functools

import jax
import jax.numpy as jnp
import numpy as np
from jax import lax
from jax.experimental import pallas as pl
from jax.experimental.pallas import tpu as pltpu
from jax.experimental.pallas import tpu_sc as plsc

_K_OUT = 1000
_THR = 0.7
_PRE = 6000
_CAP = 6144
_NB = _CAP // 128
_N_OUT_PAD = 1024


def _tau_body(bits_ref, out_ref, *, pre):
    bits = bits_ref[...]

    def step(i, tau):
        cand = tau | jax.lax.shift_left(jnp.int32(1), 30 - i)
        cnt = jnp.sum((bits >= cand).astype(jnp.int32))
        return jnp.where(cnt >= pre, cand, tau)

    tau = jax.lax.fori_loop(0, 31, step, jnp.int32(0))
    cgt = jnp.sum((bits > tau).astype(jnp.int32))
    rows = jax.lax.broadcasted_iota(jnp.int32, (8, 128), 0)
    out_ref[...] = jnp.where(rows == 0, tau, cgt)


def _stage_a(bits2d, pre):
    return pl.pallas_call(
        functools.partial(_tau_body, pre=pre),
        out_shape=jax.ShapeDtypeStruct((8, 128), jnp.int32),
    )(bits2d)


_BLK = 512
_RPB = _BLK // 128


def _tc_compact_body(bits_ref, tau_ref, rows_ref, obits_ref, oidx_ref,
                     orows_ref, comp_ref, *, n, npad, cap, pre):
    f32 = jnp.float32
    i32 = jnp.int32
    hp = jax.lax.Precision.HIGHEST
    tau = tau_ref[0, 0]
    cgt = tau_ref[1, 0]
    quota_f = (pre - cgt).astype(f32)
    nblk = npad // _BLK

    lt128 = (jax.lax.broadcasted_iota(i32, (128, 128), 0)
             <= jax.lax.broadcasted_iota(i32, (128, 128), 1)).astype(f32)
    slt8 = (jax.lax.broadcasted_iota(i32, (8, 8), 0)
            < jax.lax.broadcasted_iota(i32, (8, 8), 1)).astype(f32)
    slot_col = jax.lax.broadcasted_iota(i32, (_BLK, 1), 0)
    sub128 = jax.lax.broadcasted_iota(i32, (_RPB, 128), 0)
    lane128 = jax.lax.broadcasted_iota(i32, (_RPB, 128), 1)

    def flat_prefix_excl(m):
        incl = jnp.dot(m, lt128, preferred_element_type=f32, precision=hp)
        rt = incl[:, 127:128]
        rt8 = jnp.concatenate([rt, jnp.zeros((8 - _RPB, 1), f32)], axis=0)
        offs8 = jnp.transpose(
            jnp.dot(jnp.transpose(rt8, (1, 0)), slt8,
                    preferred_element_type=f32, precision=hp), (1, 0))
        return incl - m + offs8[0:_RPB], incl[_RPB - 1:_RPB, 127:128]

    def blk_step(b, carry):
        cursor, eqcnt = carry
        bb = bits_ref[pl.ds(b * _RPB, _RPB), :]
        gi = b * _BLK + sub128 * 128 + lane128
        inb = gi < n
        mgt = (bb > tau) & inb
        meq = (bb == tau) & inb
        eq_excl, _ = flat_prefix_excl(meq.astype(f32))
        eq_sel = meq & (eq_excl < (quota_f - eqcnt))
        sel = (mgt | eq_sel).astype(f32)
        pos_excl, _ = flat_prefix_excl(sel)
        cnt = jnp.sum(sel)
        pos_row = jnp.reshape(jnp.where(sel > 0.5, pos_excl, -1.0), (1, _BLK))
        p_mat = (slot_col.astype(f32) == pos_row).astype(f32)
        vt = jnp.transpose(rows_ref[0:16, pl.ds(b * _BLK, _BLK)], (1, 0))
        blkout = jnp.dot(p_mat, vt, preferred_element_type=f32, precision=hp)
        comp_ref[pl.ds(cursor, _BLK), :] = blkout
        return (cursor + cnt.astype(i32), eqcnt + jnp.sum(eq_sel.astype(f32)))

    jax.lax.fori_loop(0, nblk, blk_step, (jnp.int32(0), jnp.float32(0.0)))

    valid_c1 = jax.lax.broadcasted_iota(i32, (cap, 1), 0) < pre
    valid_c8 = jax.lax.broadcasted_iota(i32, (cap, 8), 0) < pre
    hi = comp_ref[0:cap, 8:9]
    lo = comp_ref[0:cap, 9:10]
    idxf = comp_ref[0:cap, 10:11]
    obits_ref[...] = jnp.where(
        valid_c1, hi.astype(i32) * 4096 + lo.astype(i32), -1)
    oidx_ref[...] = jnp.where(
        valid_c1, idxf.astype(i32),
        n + jax.lax.broadcasted_iota(i32, (cap, 1), 0))
    orows_ref[...] = jnp.where(valid_c8, comp_ref[0:cap, 0:8], 0.0)


def _stage_b_tc(bits2d, taug, rows_in, *, n, npad, cap, pre):
    f32 = jnp.float32
    i32 = jnp.int32
    return pl.pallas_call(
        functools.partial(_tc_compact_body, n=n, npad=npad, cap=cap, pre=pre),
        out_shape=[
            jax.ShapeDtypeStruct((cap, 1), i32),
            jax.ShapeDtypeStruct((cap, 1), i32),
            jax.ShapeDtypeStruct((cap, 8), f32),
        ],
        scratch_shapes=[pltpu.VMEM((cap + _BLK, 16), f32)],
    )(bits2d, taug, rows_in)


def _decode_rows(ad):
    ay1, ax1, ay2, ax2 = ad[0:1], ad[1:2], ad[2:3], ad[3:4]
    dy, dx = ad[4:5] * 0.1, ad[5:6] * 0.1
    dh, dw = ad[6:7] * 0.2, ad[7:8] * 0.2
    h = ay2 - ay1
    w = ax2 - ax1
    cy = ay1 + 0.5 * h + dy * h
    cx = ax1 + 0.5 * w + dx * w
    nh = h * jnp.exp(dh)
    nw = w * jnp.exp(dw)
    y1 = cy - 0.5 * nh
    x1 = cx - 0.5 * nw
    y2 = y1 + nh
    x2 = x1 + nw
    out = jnp.concatenate([y1, x1, y2, x2], axis=0)
    return jnp.clip(out, 0.0, 1.0)


def _stage_c_body(bits_row_ref, idx_row_ref, bits_col_ref, idx_col_ref,
                  ad_row_ref, ad_col_ref, out_ref,
                  rank_col_ref, scol_ref, srow_ref, keep_ref,
                  *, cap, nb, nvalid, nout_pad, thr):
    f32 = jnp.float32

    vrow = _decode_rows(ad_row_ref[...])
    adc = ad_col_ref[...]
    vcol = jnp.transpose(
        _decode_rows(jnp.transpose(adc, (1, 0))), (1, 0))

    brow = bits_row_ref[...]
    irow = idx_row_ref[...]

    def rank_step(b, rank_row):
        sl = pl.ds(b * 128, 128)
        bcol = bits_col_ref[sl, 0:1]
        icol = idx_col_ref[sl, 0:1]
        beats = (bcol > brow) | ((bcol == brow) & (icol < irow))
        m = beats.astype(jnp.int32)
        rank_col_ref[sl, 0:1] = (cap - 1) - jnp.sum(m, axis=1, keepdims=True)
        return rank_row + jnp.sum(m, axis=0, keepdims=True)

    rank_row = jax.lax.fori_loop(0, nb, rank_step, jnp.zeros((1, cap), jnp.int32))

    sub_iota = jax.lax.broadcasted_iota(jnp.int32, (128, 1), 0)
    lane_iota128 = jax.lax.broadcasted_iota(jnp.int32, (1, 128), 1)

    def perm_step(b, carry):
        sl = pl.ds(b * 128, 128)
        p_b = (rank_row == (b * 128 + sub_iota)).astype(f32)
        scol_ref[sl, 0:4] = jnp.dot(p_b, vcol, preferred_element_type=f32, precision=jax.lax.Precision.HIGHEST)
        pt_b = (rank_col_ref[...] == (b * 128 + lane_iota128)).astype(f32)
        srow_ref[0:4, sl] = jnp.dot(vrow, pt_b, preferred_element_type=f32, precision=jax.lax.Precision.HIGHEST)
        return carry

    jax.lax.fori_loop(0, nb, perm_step, 0)

    sr = srow_ref[0:4, :]
    srow_ref[4:5, :] = (sr[2:3] - sr[0:1]) * (sr[3:4] - sr[1:2])
    sc4 = scol_ref[:, 0:4]
    scol_ref[:, 4:5] = (sc4[:, 2:3] - sc4[:, 0:1]) * (sc4[:, 3:4] - sc4[:, 1:2])

    lane_iota_cap = jax.lax.broadcasted_iota(jnp.int32, (1, cap), 1)
    keep_ref[...] = (lane_iota_cap < nvalid).astype(f32)

    def nms_step(b, carry):
        sl = pl.ds(b * 128, 128)
        by1 = scol_ref[sl, 0:1]
        bx1 = scol_ref[sl, 1:2]
        by2 = scol_ref[sl, 2:3]
        bx2 = scol_ref[sl, 3:4]
        barea = scol_ref[sl, 4:5]
        y1r = srow_ref[0:1, :]
        x1r = srow_ref[1:2, :]
        y2r = srow_ref[2:3, :]
        x2r = srow_ref[3:4, :]
        arear = srow_ref[4:5, :]

        ry1 = srow_ref[0:1, sl]
        rx1 = srow_ref[1:2, sl]
        ry2 = srow_ref[2:3, sl]
        rx2 = srow_ref[3:4, sl]
        rarea = srow_ref[4:5, sl]
        ih = jnp.maximum(jnp.minimum(by2, ry2) - jnp.maximum(by1, ry1), 0.0)
        iw = jnp.maximum(jnp.minimum(bx2, rx2) - jnp.maximum(bx1, rx1), 0.0)
        inter = ih * iw
        iou = inter / jnp.maximum(barea + rarea - inter, 1e-8)
        s_mat = ((iou > thr) & (sub_iota < lane_iota128)).astype(f32)

        kb_init = keep_ref[0:1, sl]

        def fp_cond(c):
            return c[1]

        def fp_body(c):
            kb, _ = c
            kb_col = jnp.transpose(kb, (1, 0))
            supp = jnp.max(s_mat * kb_col, axis=0, keepdims=True)
            kb_new = kb_init * (1.0 - supp)
            return (kb_new, jnp.any(kb_new != kb))

        kb, _ = jax.lax.while_loop(fp_cond, fp_body, (kb_init, jnp.bool_(True)))
        keep_ref[0:1, sl] = kb

        kb_col = jnp.transpose(kb, (1, 0))
        fih = jnp.maximum(jnp.minimum(by2, y2r) - jnp.maximum(by1, y1r), 0.0)
        fiw = jnp.maximum(jnp.minimum(bx2, x2r) - jnp.maximum(bx1, x1r), 0.0)
        finter = fih * fiw
        fiou = finter / jnp.maximum(barea + arear - finter, 1e-8)
        su = (fiou > thr).astype(f32) * kb_col
        supp_row = jnp.max(su, axis=0, keepdims=True)
        later = (lane_iota_cap >= (b + 1) * 128).astype(f32)
        keep_ref[...] = keep_ref[...] * (1.0 - supp_row * later)
        return carry

    jax.lax.fori_loop(0, nb, nms_step, 0)

    keep = keep_ref[...]
    kb48 = jnp.reshape(keep, (nb, 128))
    lt128 = (jax.lax.broadcasted_iota(jnp.int32, (128, 128), 0)
             <= jax.lax.broadcasted_iota(jnp.int32, (128, 128), 1)).astype(f32)
    incl = jnp.dot(kb48, lt128, preferred_element_type=f32, precision=jax.lax.Precision.HIGHEST)
    rowtot = incl[:, 127:128]
    slt = (jax.lax.broadcasted_iota(jnp.int32, (nb, nb), 0)
           < jax.lax.broadcasted_iota(jnp.int32, (nb, nb), 1)).astype(f32)
    rowoff = jnp.transpose(
        jnp.dot(jnp.transpose(rowtot, (1, 0)), slt, preferred_element_type=f32, precision=jax.lax.Precision.HIGHEST),
        (1, 0))
    outpos = jnp.reshape(incl + rowoff - 1.0, (1, cap))
    scol4 = scol_ref[:, 0:4]
    sub_f = sub_iota.astype(f32)
    for r in range(nout_pad // 128):
        p2 = ((outpos == (r * 128 + sub_f)) & (keep > 0.5)).astype(f32)
        out_ref[r * 128:(r + 1) * 128, 0:4] = jnp.dot(
            p2, scol4, preferred_element_type=f32, precision=jax.lax.Precision.HIGHEST)


def _stage_c(cand_bits, cand_idx, cand_rows, *, cap, nb, nvalid, nout_pad, thr):
    bits_row = cand_bits.reshape(1, cap)
    idx_row = cand_idx.reshape(1, cap)
    bits_col = cand_bits.reshape(cap, 1)
    idx_col = cand_idx.reshape(cap, 1)
    ad_row = jnp.transpose(cand_rows, (1, 0))
    ad_col = cand_rows
    f32 = jnp.float32
    return pl.pallas_call(
        functools.partial(_stage_c_body, cap=cap, nb=nb, nvalid=nvalid,
                          nout_pad=nout_pad, thr=thr),
        out_shape=jax.ShapeDtypeStruct((nout_pad, 4), f32),
        scratch_shapes=[
            pltpu.VMEM((cap, 1), jnp.int32),
            pltpu.VMEM((cap, 8), f32),
            pltpu.VMEM((8, cap), f32),
            pltpu.VMEM((1, cap), f32),
        ],
    )(bits_row, idx_row, bits_col, idx_col, ad_row, ad_col)


def kernel(rpn_class, rpn_bbox, anchors):
    n = rpn_class.shape[1]
    npad = 262144
    f32 = jnp.float32
    scores = rpn_class[0, :, 1]
    bits = lax.bitcast_convert_type(scores, jnp.int32)
    bits_pad = jnp.pad(bits, (0, npad - n))
    bits2d = bits_pad.reshape(npad // 128, 128)
    taug = _stage_a(bits2d, _PRE)
    hi = jax.lax.shift_right_logical(bits_pad, 12).astype(f32)
    lo = (bits_pad & 4095).astype(f32)
    idxf = jnp.arange(npad, dtype=f32)
    at_ = jnp.transpose(jnp.pad(anchors[0], ((0, npad - n), (0, 0))), (1, 0))
    dt_ = jnp.transpose(jnp.pad(rpn_bbox[0], ((0, npad - n), (0, 0))), (1, 0))
    rows_in = jnp.concatenate(
        [at_, dt_, hi[None], lo[None], idxf[None],
         jnp.zeros((5, npad), f32)], axis=0)
    obits, oidx, orows = _stage_b_tc(bits2d, taug, rows_in,
                                     n=n, npad=npad, cap=_CAP, pre=_PRE)
    out = _stage_c(obits.reshape(_CAP), oidx.reshape(_CAP), orows,
                   cap=_CAP, nb=_NB, nvalid=_PRE, nout_pad=_N_OUT_PAD,
                   thr=_THR)
    return out[:_K_OUT].reshape(1, _K_OUT, 4)


# speedup vs baseline: 22.2939x; 1.0810x over previous
import functools

import jax
import jax.numpy as jnp
import numpy as np
from jax import lax
from jax.experimental import pallas as pl
from jax.experimental.pallas import tpu as pltpu
from jax.experimental.pallas import tpu_sc as plsc

_K_OUT = 1000
_THR = 0.7
_PRE = 6000
_CAP = 6144
_NB = _CAP // 128
_N_OUT_PAD = 1024


def _tau_body(bits_ref, out_ref, *, pre):
    bits = bits_ref[...]

    def step(i, tau):
        cand = tau | jax.lax.shift_left(jnp.int32(1), 30 - i)
        cnt = jnp.sum((bits >= cand).astype(jnp.int32))
        return jnp.where(cnt >= pre, cand, tau)

    tau = jax.lax.fori_loop(0, 31, step, jnp.int32(0))
    cgt = jnp.sum((bits > tau).astype(jnp.int32))
    rows = jax.lax.broadcasted_iota(jnp.int32, (8, 128), 0)
    out_ref[...] = jnp.where(rows == 0, tau, cgt)


def _stage_a(bits2d, pre):
    return pl.pallas_call(
        functools.partial(_tau_body, pre=pre),
        out_shape=jax.ShapeDtypeStruct((8, 128), jnp.int32),
    )(bits2d)


_BLK = 512
_RPB = _BLK // 128


def _tc_compact_body(bits_ref, tau_ref, rows_ref, obits_ref, oidx_ref,
                     orows_ref, comp_ref, *, n, npad, cap, pre):
    f32 = jnp.float32
    i32 = jnp.int32
    hp = jax.lax.Precision.HIGHEST
    tau = tau_ref[0, 0]
    cgt = tau_ref[1, 0]
    quota_f = (pre - cgt).astype(f32)
    nblk = npad // _BLK

    slot_col = jax.lax.broadcasted_iota(i32, (_BLK, 1), 0)
    sub128 = jax.lax.broadcasted_iota(i32, (_RPB, 128), 0)
    lane128 = jax.lax.broadcasted_iota(i32, (_RPB, 128), 1)

    def flat_prefix_excl(m):
        incl = m
        d = 1
        while d < 128:
            incl = incl + jnp.concatenate(
                [jnp.zeros((_RPB, d), f32), incl[:, : 128 - d]], axis=1)
            d *= 2
        z = jnp.zeros((1, 128), f32)
        offs = [z]
        acc = incl[0:1, 127:128]
        for r in range(1, _RPB):
            offs.append(z + acc)
            acc = acc + incl[r:r + 1, 127:128]
        offs_mat = jnp.concatenate(offs, axis=0)
        return incl - m + offs_mat, acc

    def blk_step(b, carry):
        cursor, eqcnt = carry
        bb = bits_ref[pl.ds(b * _RPB, _RPB), :]
        gi = b * _BLK + sub128 * 128 + lane128
        inb = gi < n
        mgt = (bb > tau) & inb
        meq = (bb == tau) & inb
        eq_excl, _ = flat_prefix_excl(meq.astype(f32))
        eq_sel = meq & (eq_excl < (quota_f - eqcnt))
        sel = (mgt | eq_sel).astype(f32)
        pos_excl, _ = flat_prefix_excl(sel)
        cnt = jnp.sum(sel)
        pos_row = jnp.reshape(jnp.where(sel > 0.5, pos_excl, -1.0), (1, _BLK))
        p_mat = (slot_col.astype(f32) == pos_row).astype(f32)
        vt = jnp.transpose(rows_ref[0:16, pl.ds(b * _BLK, _BLK)], (1, 0))
        blkout = jnp.dot(p_mat, vt, preferred_element_type=f32, precision=hp)
        comp_ref[pl.ds(cursor, _BLK), :] = blkout
        return (cursor + cnt.astype(i32), eqcnt + jnp.sum(eq_sel.astype(f32)))

    jax.lax.fori_loop(0, nblk, blk_step, (jnp.int32(0), jnp.float32(0.0)))

    valid_c1 = jax.lax.broadcasted_iota(i32, (cap, 1), 0) < pre
    valid_c8 = jax.lax.broadcasted_iota(i32, (cap, 8), 0) < pre
    hi = comp_ref[0:cap, 8:9]
    lo = comp_ref[0:cap, 9:10]
    idxf = comp_ref[0:cap, 10:11]
    obits_ref[...] = jnp.where(
        valid_c1, hi.astype(i32) * 4096 + lo.astype(i32), -1)
    oidx_ref[...] = jnp.where(
        valid_c1, idxf.astype(i32),
        n + jax.lax.broadcasted_iota(i32, (cap, 1), 0))
    orows_ref[...] = jnp.where(valid_c8, comp_ref[0:cap, 0:8], 0.0)


def _stage_b_tc(bits2d, taug, rows_in, *, n, npad, cap, pre):
    f32 = jnp.float32
    i32 = jnp.int32
    return pl.pallas_call(
        functools.partial(_tc_compact_body, n=n, npad=npad, cap=cap, pre=pre),
        out_shape=[
            jax.ShapeDtypeStruct((cap, 1), i32),
            jax.ShapeDtypeStruct((cap, 1), i32),
            jax.ShapeDtypeStruct((cap, 8), f32),
        ],
        scratch_shapes=[pltpu.VMEM((cap + _BLK, 16), f32)],
    )(bits2d, taug, rows_in)


def _decode_rows(ad):
    ay1, ax1, ay2, ax2 = ad[0:1], ad[1:2], ad[2:3], ad[3:4]
    dy, dx = ad[4:5] * 0.1, ad[5:6] * 0.1
    dh, dw = ad[6:7] * 0.2, ad[7:8] * 0.2
    h = ay2 - ay1
    w = ax2 - ax1
    cy = ay1 + 0.5 * h + dy * h
    cx = ax1 + 0.5 * w + dx * w
    nh = h * jnp.exp(dh)
    nw = w * jnp.exp(dw)
    y1 = cy - 0.5 * nh
    x1 = cx - 0.5 * nw
    y2 = y1 + nh
    x2 = x1 + nw
    out = jnp.concatenate([y1, x1, y2, x2], axis=0)
    return jnp.clip(out, 0.0, 1.0)


def _stage_c_body(bits_row_ref, idx_row_ref, bits_col_ref, idx_col_ref,
                  ad_row_ref, ad_col_ref, out_ref,
                  rank_col_ref, scol_ref, srow_ref, keep_ref,
                  *, cap, nb, nvalid, nout_pad, thr):
    f32 = jnp.float32

    vrow = _decode_rows(ad_row_ref[...])
    adc = ad_col_ref[...]
    vcol = jnp.transpose(
        _decode_rows(jnp.transpose(adc, (1, 0))), (1, 0))

    brow = bits_row_ref[...]
    irow = idx_row_ref[...]

    def rank_step(b, rank_row):
        sl = pl.ds(b * 128, 128)
        bcol = bits_col_ref[sl, 0:1]
        icol = idx_col_ref[sl, 0:1]
        beats = (bcol > brow) | ((bcol == brow) & (icol < irow))
        m = beats.astype(jnp.int32)
        rank_col_ref[sl, 0:1] = (cap - 1) - jnp.sum(m, axis=1, keepdims=True)
        return rank_row + jnp.sum(m, axis=0, keepdims=True)

    rank_row = jax.lax.fori_loop(0, nb, rank_step, jnp.zeros((1, cap), jnp.int32))

    sub_iota = jax.lax.broadcasted_iota(jnp.int32, (128, 1), 0)
    lane_iota128 = jax.lax.broadcasted_iota(jnp.int32, (1, 128), 1)

    def perm_step(b, carry):
        sl = pl.ds(b * 128, 128)
        p_b = (rank_row == (b * 128 + sub_iota)).astype(f32)
        scol_ref[sl, 0:4] = jnp.dot(p_b, vcol, preferred_element_type=f32, precision=jax.lax.Precision.HIGHEST)
        pt_b = (rank_col_ref[...] == (b * 128 + lane_iota128)).astype(f32)
        srow_ref[0:4, sl] = jnp.dot(vrow, pt_b, preferred_element_type=f32, precision=jax.lax.Precision.HIGHEST)
        return carry

    jax.lax.fori_loop(0, nb, perm_step, 0)

    sr = srow_ref[0:4, :]
    srow_ref[4:5, :] = (sr[2:3] - sr[0:1]) * (sr[3:4] - sr[1:2])
    sc4 = scol_ref[:, 0:4]
    scol_ref[:, 4:5] = (sc4[:, 2:3] - sc4[:, 0:1]) * (sc4[:, 3:4] - sc4[:, 1:2])

    lane_iota_cap = jax.lax.broadcasted_iota(jnp.int32, (1, cap), 1)
    keep_ref[...] = (lane_iota_cap < nvalid).astype(f32)

    def make_nms_step(lo):
        w = cap - lo

        def nms_step(b, carry):
            sl = pl.ds(b * 128, 128)
            by1 = scol_ref[sl, 0:1]
            bx1 = scol_ref[sl, 1:2]
            by2 = scol_ref[sl, 2:3]
            bx2 = scol_ref[sl, 3:4]
            barea = scol_ref[sl, 4:5]

            ry1 = srow_ref[0:1, sl]
            rx1 = srow_ref[1:2, sl]
            ry2 = srow_ref[2:3, sl]
            rx2 = srow_ref[3:4, sl]
            rarea = srow_ref[4:5, sl]
            ih = jnp.maximum(jnp.minimum(by2, ry2) - jnp.maximum(by1, ry1), 0.0)
            iw = jnp.maximum(jnp.minimum(bx2, rx2) - jnp.maximum(bx1, rx1), 0.0)
            inter = ih * iw
            iou = inter / jnp.maximum(barea + rarea - inter, 1e-8)
            s_mat = ((iou > thr) & (sub_iota < lane_iota128)).astype(f32)

            kb_init = keep_ref[0:1, sl]

            def fp_cond(c):
                return c[1]

            def fp_body(c):
                kb, _ = c
                kb_col = jnp.transpose(kb, (1, 0))
                supp = jnp.max(s_mat * kb_col, axis=0, keepdims=True)
                kb_new = kb_init * (1.0 - supp)
                return (kb_new, jnp.any(kb_new != kb))

            kb, _ = jax.lax.while_loop(fp_cond, fp_body, (kb_init, jnp.bool_(True)))
            keep_ref[0:1, sl] = kb

            kb_col = jnp.transpose(kb, (1, 0))
            y1t = srow_ref[0:1, lo:cap]
            x1t = srow_ref[1:2, lo:cap]
            y2t = srow_ref[2:3, lo:cap]
            x2t = srow_ref[3:4, lo:cap]
            areat = srow_ref[4:5, lo:cap]
            fih = jnp.maximum(jnp.minimum(by2, y2t) - jnp.maximum(by1, y1t), 0.0)
            fiw = jnp.maximum(jnp.minimum(bx2, x2t) - jnp.maximum(bx1, x1t), 0.0)
            finter = fih * fiw
            fiou = finter / jnp.maximum(barea + areat - finter, 1e-8)
            su = (fiou > thr).astype(f32) * kb_col
            supp_row = jnp.max(su, axis=0, keepdims=True)
            later = (jax.lax.broadcasted_iota(jnp.int32, (1, w), 1) + lo
                     >= (b + 1) * 128).astype(f32)
            keep_ref[0:1, lo:cap] = keep_ref[0:1, lo:cap] * (1.0 - supp_row * later)
            return carry

        return nms_step

    nseg = 4
    bps = nb // nseg
    for seg in range(nseg):
        jax.lax.fori_loop(seg * bps, (seg + 1) * bps,
                          make_nms_step(seg * bps * 128), 0)

    keep = keep_ref[...]
    kb48 = jnp.reshape(keep, (nb, 128))
    lt128 = (jax.lax.broadcasted_iota(jnp.int32, (128, 128), 0)
             <= jax.lax.broadcasted_iota(jnp.int32, (128, 128), 1)).astype(f32)
    incl = jnp.dot(kb48, lt128, preferred_element_type=f32, precision=jax.lax.Precision.HIGHEST)
    rowtot = incl[:, 127:128]
    slt = (jax.lax.broadcasted_iota(jnp.int32, (nb, nb), 0)
           < jax.lax.broadcasted_iota(jnp.int32, (nb, nb), 1)).astype(f32)
    rowoff = jnp.transpose(
        jnp.dot(jnp.transpose(rowtot, (1, 0)), slt, preferred_element_type=f32, precision=jax.lax.Precision.HIGHEST),
        (1, 0))
    outpos = jnp.reshape(incl + rowoff - 1.0, (1, cap))
    scol4 = scol_ref[:, 0:4]
    sub_f = sub_iota.astype(f32)
    for r in range(nout_pad // 128):
        p2 = ((outpos == (r * 128 + sub_f)) & (keep > 0.5)).astype(f32)
        out_ref[r * 128:(r + 1) * 128, 0:4] = jnp.dot(
            p2, scol4, preferred_element_type=f32, precision=jax.lax.Precision.HIGHEST)


def _stage_c(cand_bits, cand_idx, cand_rows, *, cap, nb, nvalid, nout_pad, thr):
    bits_row = cand_bits.reshape(1, cap)
    idx_row = cand_idx.reshape(1, cap)
    bits_col = cand_bits.reshape(cap, 1)
    idx_col = cand_idx.reshape(cap, 1)
    ad_row = jnp.transpose(cand_rows, (1, 0))
    ad_col = cand_rows
    f32 = jnp.float32
    return pl.pallas_call(
        functools.partial(_stage_c_body, cap=cap, nb=nb, nvalid=nvalid,
                          nout_pad=nout_pad, thr=thr),
        out_shape=jax.ShapeDtypeStruct((nout_pad, 4), f32),
        scratch_shapes=[
            pltpu.VMEM((cap, 1), jnp.int32),
            pltpu.VMEM((cap, 8), f32),
            pltpu.VMEM((8, cap), f32),
            pltpu.VMEM((1, cap), f32),
        ],
    )(bits_row, idx_row, bits_col, idx_col, ad_row, ad_col)


def kernel(rpn_class, rpn_bbox, anchors):
    n = rpn_class.shape[1]
    npad = 262144
    f32 = jnp.float32
    scores = rpn_class[0, :, 1]
    bits = lax.bitcast_convert_type(scores, jnp.int32)
    bits_pad = jnp.pad(bits, (0, npad - n))
    bits2d = bits_pad.reshape(npad // 128, 128)
    taug = _stage_a(bits2d, _PRE)
    hi = jax.lax.shift_right_logical(bits_pad, 12).astype(f32)
    lo = (bits_pad & 4095).astype(f32)
    idxf = jnp.arange(npad, dtype=f32)
    at_ = jnp.transpose(jnp.pad(anchors[0], ((0, npad - n), (0, 0))), (1, 0))
    dt_ = jnp.transpose(jnp.pad(rpn_bbox[0], ((0, npad - n), (0, 0))), (1, 0))
    rows_in = jnp.concatenate(
        [at_, dt_, hi[None], lo[None], idxf[None],
         jnp.zeros((5, npad), f32)], axis=0)
    obits, oidx, orows = _stage_b_tc(bits2d, taug, rows_in,
                                     n=n, npad=npad, cap=_CAP, pre=_PRE)
    out = _stage_c(obits.reshape(_CAP), oidx.reshape(_CAP), orows,
                   cap=_CAP, nb=_NB, nvalid=_PRE, nout_pad=_N_OUT_PAD,
                   thr=_THR)
    return out[:_K_OUT].reshape(1, _K_OUT, 4)


# speedup vs baseline: 22.4367x; 1.0064x over previous
import functools

import jax
import jax.numpy as jnp
import numpy as np
from jax import lax
from jax.experimental import pallas as pl
from jax.experimental.pallas import tpu as pltpu
from jax.experimental.pallas import tpu_sc as plsc

_K_OUT = 1000
_THR = 0.7
_PRE = 6000
_CAP = 6144
_NB = _CAP // 128
_N_OUT_PAD = 1024


def _tau_body(bits_ref, out_ref, *, pre):
    bits = bits_ref[...]

    def step(i, tau):
        cand = tau | jax.lax.shift_left(jnp.int32(1), 30 - i)
        cnt = jnp.sum((bits >= cand).astype(jnp.int32))
        return jnp.where(cnt >= pre, cand, tau)

    tau = jax.lax.fori_loop(0, 31, step, jnp.int32(0))
    cgt = jnp.sum((bits > tau).astype(jnp.int32))
    rows = jax.lax.broadcasted_iota(jnp.int32, (8, 128), 0)
    out_ref[...] = jnp.where(rows == 0, tau, cgt)


def _stage_a(bits2d, pre):
    return pl.pallas_call(
        functools.partial(_tau_body, pre=pre),
        out_shape=jax.ShapeDtypeStruct((8, 128), jnp.int32),
    )(bits2d)


_BLK = 512
_RPB = _BLK // 128


def _tc_compact_body(bits_ref, tau_ref, rows_ref, obits_ref, oidx_ref,
                     orows_ref, comp_ref, *, n, npad, cap, pre):
    f32 = jnp.float32
    i32 = jnp.int32
    hp = jax.lax.Precision.HIGHEST
    tau = tau_ref[0, 0]
    cgt = tau_ref[1, 0]
    quota_f = (pre - cgt).astype(f32)
    nblk = npad // _BLK

    slot_col = jax.lax.broadcasted_iota(i32, (_BLK, 1), 0)
    sub128 = jax.lax.broadcasted_iota(i32, (_RPB, 128), 0)
    lane128 = jax.lax.broadcasted_iota(i32, (_RPB, 128), 1)

    def flat_prefix_excl(m):
        incl = m
        d = 1
        while d < 128:
            incl = incl + jnp.concatenate(
                [jnp.zeros((_RPB, d), f32), incl[:, : 128 - d]], axis=1)
            d *= 2
        z = jnp.zeros((1, 128), f32)
        offs = [z]
        acc = incl[0:1, 127:128]
        for r in range(1, _RPB):
            offs.append(z + acc)
            acc = acc + incl[r:r + 1, 127:128]
        offs_mat = jnp.concatenate(offs, axis=0)
        return incl - m + offs_mat, acc

    def blk_step(b, carry):
        cursor, eqcnt = carry
        bb = bits_ref[pl.ds(b * _RPB, _RPB), :]
        gi = b * _BLK + sub128 * 128 + lane128
        inb = gi < n
        mgt = (bb > tau) & inb
        meq = (bb == tau) & inb
        eq_excl, _ = flat_prefix_excl(meq.astype(f32))
        eq_sel = meq & (eq_excl < (quota_f - eqcnt))
        sel = (mgt | eq_sel).astype(f32)
        pos_excl, _ = flat_prefix_excl(sel)
        cnt = jnp.sum(sel)
        pos_row = jnp.reshape(jnp.where(sel > 0.5, pos_excl, -1.0), (1, _BLK))
        p_mat = (slot_col.astype(f32) == pos_row).astype(f32)
        vt = jnp.transpose(rows_ref[0:11, pl.ds(b * _BLK, _BLK)], (1, 0))
        blkout = jnp.dot(p_mat, vt, preferred_element_type=f32, precision=hp)
        comp_ref[pl.ds(cursor, _BLK), :] = blkout
        return (cursor + cnt.astype(i32), eqcnt + jnp.sum(eq_sel.astype(f32)))

    jax.lax.fori_loop(0, nblk, blk_step, (jnp.int32(0), jnp.float32(0.0)))

    valid_c1 = jax.lax.broadcasted_iota(i32, (cap, 1), 0) < pre
    valid_c8 = jax.lax.broadcasted_iota(i32, (cap, 8), 0) < pre
    hi = comp_ref[0:cap, 8:9]
    lo = comp_ref[0:cap, 9:10]
    idxf = comp_ref[0:cap, 10:11]
    obits_ref[...] = jnp.where(
        valid_c1, hi.astype(i32) * 4096 + lo.astype(i32), -1)
    oidx_ref[...] = jnp.where(
        valid_c1, idxf.astype(i32),
        n + jax.lax.broadcasted_iota(i32, (cap, 1), 0))
    orows_ref[...] = jnp.where(valid_c8, comp_ref[0:cap, 0:8], 0.0)


def _stage_b_tc(bits2d, taug, rows_in, *, n, npad, cap, pre):
    f32 = jnp.float32
    i32 = jnp.int32
    return pl.pallas_call(
        functools.partial(_tc_compact_body, n=n, npad=npad, cap=cap, pre=pre),
        out_shape=[
            jax.ShapeDtypeStruct((cap, 1), i32),
            jax.ShapeDtypeStruct((cap, 1), i32),
            jax.ShapeDtypeStruct((cap, 8), f32),
        ],
        scratch_shapes=[pltpu.VMEM((cap + _BLK, 11), f32)],
    )(bits2d, taug, rows_in)


def _decode_rows(ad):
    ay1, ax1, ay2, ax2 = ad[0:1], ad[1:2], ad[2:3], ad[3:4]
    dy, dx = ad[4:5] * 0.1, ad[5:6] * 0.1
    dh, dw = ad[6:7] * 0.2, ad[7:8] * 0.2
    h = ay2 - ay1
    w = ax2 - ax1
    cy = ay1 + 0.5 * h + dy * h
    cx = ax1 + 0.5 * w + dx * w
    nh = h * jnp.exp(dh)
    nw = w * jnp.exp(dw)
    y1 = cy - 0.5 * nh
    x1 = cx - 0.5 * nw
    y2 = y1 + nh
    x2 = x1 + nw
    out = jnp.concatenate([y1, x1, y2, x2], axis=0)
    return jnp.clip(out, 0.0, 1.0)


def _stage_c_body(bits_row_ref, idx_row_ref, bits_col_ref, idx_col_ref,
                  ad_row_ref, ad_col_ref, out_ref,
                  rank_col_ref, scol_ref, srow_ref, keep_ref,
                  *, cap, nb, nvalid, nout_pad, thr):
    f32 = jnp.float32

    vrow = _decode_rows(ad_row_ref[...])
    adc = ad_col_ref[...]
    vcol = jnp.transpose(
        _decode_rows(jnp.transpose(adc, (1, 0))), (1, 0))

    brow = bits_row_ref[...]
    irow = idx_row_ref[...]

    def make_rank_step(lo):
        w = cap - lo

        def rank_step(b, rank_row):
            sl = pl.ds(b * 128, 128)
            bcol = bits_col_ref[sl, 0:1]
            icol = idx_col_ref[sl, 0:1]
            brow_b = bits_row_ref[0:1, sl]
            irow_b = idx_row_ref[0:1, sl]
            m_bb = ((bcol > brow_b) | ((bcol == brow_b) & (icol < irow_b))
                    ).astype(jnp.int32)
            intra = 127 - jnp.sum(m_bb, axis=1, keepdims=True)
            brow_t = bits_row_ref[0:1, lo:cap]
            irow_t = idx_row_ref[0:1, lo:cap]
            beats = (bcol > brow_t) | ((bcol == brow_t) & (icol < irow_t))
            validj = (jax.lax.broadcasted_iota(jnp.int32, (1, w), 1) + lo
                      >= (b + 1) * 128)
            m = (beats & validj).astype(jnp.int32)
            wbs = cap - (b + 1) * 128
            rank_col_ref[sl, 0:1] = (
                rank_col_ref[sl, 0:1] + intra + wbs
                - jnp.sum(m, axis=1, keepdims=True))
            colsum = jnp.sum(m, axis=0, keepdims=True)
            if lo:
                colsum = jnp.concatenate(
                    [jnp.zeros((1, lo), jnp.int32), colsum], axis=1)
            return rank_row + colsum

        return rank_step

    rank_col_ref[...] = jnp.zeros((cap, 1), jnp.int32)
    rank_row = jnp.zeros((1, cap), jnp.int32)
    rbps = nb // 4
    for seg in range(4):
        rank_row = jax.lax.fori_loop(seg * rbps, (seg + 1) * rbps,
                                     make_rank_step(seg * rbps * 128), rank_row)
    full_col = rank_col_ref[...] + jnp.transpose(rank_row, (1, 0))
    rank_row = rank_row + jnp.transpose(rank_col_ref[...], (1, 0))
    rank_col_ref[...] = full_col

    sub_iota = jax.lax.broadcasted_iota(jnp.int32, (128, 1), 0)
    lane_iota128 = jax.lax.broadcasted_iota(jnp.int32, (1, 128), 1)

    def perm_step(b, carry):
        sl = pl.ds(b * 128, 128)
        p_b = (rank_row == (b * 128 + sub_iota)).astype(f32)
        scol_ref[sl, 0:4] = jnp.dot(p_b, vcol, preferred_element_type=f32, precision=jax.lax.Precision.HIGHEST)
        pt_b = (rank_col_ref[...] == (b * 128 + lane_iota128)).astype(f32)
        srow_ref[0:4, sl] = jnp.dot(vrow, pt_b, preferred_element_type=f32, precision=jax.lax.Precision.HIGHEST)
        return carry

    jax.lax.fori_loop(0, nb, perm_step, 0)

    sr = srow_ref[0:4, :]
    srow_ref[4:5, :] = (sr[2:3] - sr[0:1]) * (sr[3:4] - sr[1:2])
    sc4 = scol_ref[:, 0:4]
    scol_ref[:, 4:5] = (sc4[:, 2:3] - sc4[:, 0:1]) * (sc4[:, 3:4] - sc4[:, 1:2])

    lane_iota_cap = jax.lax.broadcasted_iota(jnp.int32, (1, cap), 1)
    keep_ref[...] = (lane_iota_cap < nvalid).astype(f32)

    def make_nms_step(lo):
        w = cap - lo

        def nms_step(b, carry):
            sl = pl.ds(b * 128, 128)
            by1 = scol_ref[sl, 0:1]
            bx1 = scol_ref[sl, 1:2]
            by2 = scol_ref[sl, 2:3]
            bx2 = scol_ref[sl, 3:4]
            barea = scol_ref[sl, 4:5]

            ry1 = srow_ref[0:1, sl]
            rx1 = srow_ref[1:2, sl]
            ry2 = srow_ref[2:3, sl]
            rx2 = srow_ref[3:4, sl]
            rarea = srow_ref[4:5, sl]
            ih = jnp.maximum(jnp.minimum(by2, ry2) - jnp.maximum(by1, ry1), 0.0)
            iw = jnp.maximum(jnp.minimum(bx2, rx2) - jnp.maximum(bx1, rx1), 0.0)
            inter = ih * iw
            iou = inter / jnp.maximum(barea + rarea - inter, 1e-8)
            s_mat = ((iou > thr) & (sub_iota < lane_iota128)).astype(f32)

            kb_init = keep_ref[0:1, sl]

            def fp_cond(c):
                return c[1]

            def fp_body(c):
                kb, _ = c
                kb_col = jnp.transpose(kb, (1, 0))
                supp = jnp.max(s_mat * kb_col, axis=0, keepdims=True)
                kb_new = kb_init * (1.0 - supp)
                return (kb_new, jnp.any(kb_new != kb))

            kb, _ = jax.lax.while_loop(fp_cond, fp_body, (kb_init, jnp.bool_(True)))
            keep_ref[0:1, sl] = kb

            kb_col = jnp.transpose(kb, (1, 0))
            y1t = srow_ref[0:1, lo:cap]
            x1t = srow_ref[1:2, lo:cap]
            y2t = srow_ref[2:3, lo:cap]
            x2t = srow_ref[3:4, lo:cap]
            areat = srow_ref[4:5, lo:cap]
            fih = jnp.maximum(jnp.minimum(by2, y2t) - jnp.maximum(by1, y1t), 0.0)
            fiw = jnp.maximum(jnp.minimum(bx2, x2t) - jnp.maximum(bx1, x1t), 0.0)
            finter = fih * fiw
            fiou = finter / jnp.maximum(barea + areat - finter, 1e-8)
            su = (fiou > thr).astype(f32) * kb_col
            supp_row = jnp.max(su, axis=0, keepdims=True)
            later = (jax.lax.broadcasted_iota(jnp.int32, (1, w), 1) + lo
                     >= (b + 1) * 128).astype(f32)
            keep_ref[0:1, lo:cap] = keep_ref[0:1, lo:cap] * (1.0 - supp_row * later)
            return carry

        return nms_step

    nseg = 4
    bps = nb // nseg
    for seg in range(nseg):
        jax.lax.fori_loop(seg * bps, (seg + 1) * bps,
                          make_nms_step(seg * bps * 128), 0)

    keep = keep_ref[...]
    kb48 = jnp.reshape(keep, (nb, 128))
    lt128 = (jax.lax.broadcasted_iota(jnp.int32, (128, 128), 0)
             <= jax.lax.broadcasted_iota(jnp.int32, (128, 128), 1)).astype(f32)
    incl = jnp.dot(kb48, lt128, preferred_element_type=f32, precision=jax.lax.Precision.HIGHEST)
    rowtot = incl[:, 127:128]
    slt = (jax.lax.broadcasted_iota(jnp.int32, (nb, nb), 0)
           < jax.lax.broadcasted_iota(jnp.int32, (nb, nb), 1)).astype(f32)
    rowoff = jnp.transpose(
        jnp.dot(jnp.transpose(rowtot, (1, 0)), slt, preferred_element_type=f32, precision=jax.lax.Precision.HIGHEST),
        (1, 0))
    outpos = jnp.reshape(incl + rowoff - 1.0, (1, cap))
    scol4 = scol_ref[:, 0:4]
    sub_f = sub_iota.astype(f32)
    for r in range(nout_pad // 128):
        p2 = ((outpos == (r * 128 + sub_f)) & (keep > 0.5)).astype(f32)
        out_ref[r * 128:(r + 1) * 128, 0:4] = jnp.dot(
            p2, scol4, preferred_element_type=f32, precision=jax.lax.Precision.HIGHEST)


def _stage_c(cand_bits, cand_idx, cand_rows, *, cap, nb, nvalid, nout_pad, thr):
    bits_row = cand_bits.reshape(1, cap)
    idx_row = cand_idx.reshape(1, cap)
    bits_col = cand_bits.reshape(cap, 1)
    idx_col = cand_idx.reshape(cap, 1)
    ad_row = jnp.transpose(cand_rows, (1, 0))
    ad_col = cand_rows
    f32 = jnp.float32
    return pl.pallas_call(
        functools.partial(_stage_c_body, cap=cap, nb=nb, nvalid=nvalid,
                          nout_pad=nout_pad, thr=thr),
        out_shape=jax.ShapeDtypeStruct((nout_pad, 4), f32),
        scratch_shapes=[
            pltpu.VMEM((cap, 1), jnp.int32),
            pltpu.VMEM((cap, 8), f32),
            pltpu.VMEM((8, cap), f32),
            pltpu.VMEM((1, cap), f32),
        ],
    )(bits_row, idx_row, bits_col, idx_col, ad_row, ad_col)


def kernel(rpn_class, rpn_bbox, anchors):
    n = rpn_class.shape[1]
    npad = 262144
    f32 = jnp.float32
    scores = rpn_class[0, :, 1]
    bits = lax.bitcast_convert_type(scores, jnp.int32)
    bits_pad = jnp.pad(bits, (0, npad - n))
    bits2d = bits_pad.reshape(npad // 128, 128)
    taug = _stage_a(bits2d, _PRE)
    hi = jax.lax.shift_right_logical(bits_pad, 12).astype(f32)
    lo = (bits_pad & 4095).astype(f32)
    idxf = jnp.arange(npad, dtype=f32)
    at_ = jnp.transpose(jnp.pad(anchors[0], ((0, npad - n), (0, 0))), (1, 0))
    dt_ = jnp.transpose(jnp.pad(rpn_bbox[0], ((0, npad - n), (0, 0))), (1, 0))
    rows_in = jnp.concatenate(
        [at_, dt_, hi[None], lo[None], idxf[None]], axis=0)
    obits, oidx, orows = _stage_b_tc(bits2d, taug, rows_in,
                                     n=n, npad=npad, cap=_CAP, pre=_PRE)
    out = _stage_c(obits.reshape(_CAP), oidx.reshape(_CAP), orows,
                   cap=_CAP, nb=_NB, nvalid=_PRE, nout_pad=_N_OUT_PAD,
                   thr=_THR)
    return out[:_K_OUT].reshape(1, _K_OUT, 4)
